# Initial kernel scaffold; baseline (speedup 1.0000x reference)
import jax
import jax.numpy as jnp
from jax.experimental import pallas as pl


def kernel(h, h0, adj_edge_index, adj_edge_values, W, b, ln_weight, ln_bias):
    raise NotImplementedError("write your pallas kernel here")


# speedup vs baseline: 2.5911x; 2.5911x over previous
import functools

import jax
import jax.numpy as jnp
from jax import lax
from jax.experimental import pallas as pl
from jax.experimental.pallas import tpu as pltpu
from jax.experimental.pallas import tpu_sc as plsc

N_NODES = 10000
N_EDGES = 160000
DIM = 256
HD = 128
ALPHA = 0.1
BETA = 0.5

NS = 16
EPT = N_EDGES // NS
K = 80
CHUNKS = EPT // K
ROWS_PER_TILE = N_NODES // NS
ZR = 125


def _spmm_body(hcat_hbm, src2_hbm, dst_hbm, val_hbm, out_hbm,
               acc, src_v, dst_v, val_v, rows_v, zeros_v, sem):
    c = lax.axis_index("c")
    s = lax.axis_index("s")

    z16 = jnp.zeros((16,), jnp.float32)

    def zrow(r, carry):
        for k in range(HD // 16):
            zeros_v[r, pl.ds(k * 16, 16)] = z16
        return carry

    lax.fori_loop(0, ZR, zrow, 0)
    for j in range(ROWS_PER_TILE // ZR):
        pltpu.sync_copy(zeros_v, acc.at[pl.ds(s * ROWS_PER_TILE + j * ZR, ZR)])
    plsc.subcore_barrier()

    base_e = s * EPT

    def chunk(i, carry):
        off = base_e + i * K
        pltpu.sync_copy(src2_hbm.at[pl.ds(c * N_EDGES + off, K)], src_v)
        pltpu.sync_copy(dst_hbm.at[pl.ds(off, K)], dst_v)
        pltpu.sync_copy(val_hbm.at[pl.ds(off, K)], val_v)
        pltpu.async_copy(hcat_hbm.at[src_v], rows_v, sem).wait()

        def edge(e, ecarry):
            val16 = val_v[pl.ds((e // 16) * 16, 16)]
            vb = lax.gather(
                val16, jnp.full((16, 1), e % 16, jnp.int32),
                lax.GatherDimensionNumbers(
                    offset_dims=(), collapsed_slice_dims=(0,),
                    start_index_map=(0,)),
                slice_sizes=(1,),
                mode=lax.GatherScatterMode.PROMISE_IN_BOUNDS)
            for d in range(HD // 16):
                x = rows_v[e, pl.ds(d * 16, 16)]
                rows_v[e, pl.ds(d * 16, 16)] = x * vb
            return ecarry

        lax.fori_loop(0, K, edge, 0)
        pltpu.sync_copy(rows_v, acc.at[dst_v], add=True)
        return carry

    lax.fori_loop(0, CHUNKS, chunk, 0)
    plsc.subcore_barrier()
    pltpu.sync_copy(acc.at[pl.ds(s * 624, 624)],
                    out_hbm.at[c, pl.ds(s * 624, 624)])

    @pl.when(s == 0)
    def _tail():
        pltpu.sync_copy(acc.at[pl.ds(9984, 16)],
                        out_hbm.at[c, pl.ds(9984, 16)])


_spmm = functools.partial(
    pl.kernel,
    mesh=plsc.VectorSubcoreMesh(core_axis_name="c", subcore_axis_name="s"),
    out_type=jax.ShapeDtypeStruct((2, N_NODES, HD), jnp.float32),
    scratch_types=[
        pltpu.VMEM_SHARED((N_NODES, HD), jnp.float32),
        pltpu.VMEM((K,), jnp.int32),
        pltpu.VMEM((K,), jnp.int32),
        pltpu.VMEM((K,), jnp.float32),
        pltpu.VMEM((K, HD), jnp.float32),
        pltpu.VMEM((ZR, HD), jnp.float32),
        pltpu.SemaphoreType.DMA,
    ],
)(_spmm_body)


_SQRT_HALF = 0.7071067811865476


def _erf(x):
    ax = jnp.abs(x)
    t = 1.0 / (1.0 + 0.3275911 * ax)
    poly = ((((1.061405429 * t - 1.453152027) * t + 1.421413741) * t
             - 0.284496736) * t + 0.254829592) * t
    e = 1.0 - poly * jnp.exp(-ax * ax)
    return jnp.sign(x) * e


def _dense_body(ma_ref, mb_ref, h0_ref, h_ref, wt_ref, b_ref, g_ref, be_ref,
                o_ref):
    mm = jnp.concatenate([ma_ref[...], mb_ref[...]], axis=1)
    mm = (1.0 - ALPHA) * mm + ALPHA * h0_ref[...]
    lin = jnp.dot(mm, wt_ref[...], preferred_element_type=jnp.float32)
    lin = lin + b_ref[...]
    x = (1.0 - BETA) * mm + BETA * lin
    g = 0.5 * x * (1.0 + _erf(x * _SQRT_HALF))
    y = g + h_ref[...]
    mean = jnp.mean(y, axis=1, keepdims=True)
    cen = y - mean
    var = jnp.mean(cen * cen, axis=1, keepdims=True)
    o_ref[...] = cen * lax.rsqrt(var + 1e-5) * g_ref[...] + be_ref[...]


R = 1000


def _dense(ma, mb, h0, h, wt, b2, g2, be2):
    grid = (N_NODES // R,)
    return pl.pallas_call(
        _dense_body,
        grid=grid,
        in_specs=[
            pl.BlockSpec((R, HD), lambda i: (i, 0)),
            pl.BlockSpec((R, HD), lambda i: (i, 0)),
            pl.BlockSpec((R, DIM), lambda i: (i, 0)),
            pl.BlockSpec((R, DIM), lambda i: (i, 0)),
            pl.BlockSpec((DIM, DIM), lambda i: (0, 0)),
            pl.BlockSpec((1, DIM), lambda i: (0, 0)),
            pl.BlockSpec((1, DIM), lambda i: (0, 0)),
            pl.BlockSpec((1, DIM), lambda i: (0, 0)),
        ],
        out_specs=pl.BlockSpec((R, DIM), lambda i: (i, 0)),
        out_shape=jax.ShapeDtypeStruct((N_NODES, DIM), jnp.float32),
    )(ma, mb, h0, h, wt, b2, g2, be2)


@jax.jit
def kernel(h, h0, adj_edge_index, adj_edge_values, W, b, ln_weight, ln_bias):
    src = adj_edge_index[0]
    dst = adj_edge_index[1]
    hcat = jnp.concatenate([h[:, :HD], h[:, HD:]], axis=0)
    src2 = jnp.concatenate([src, src + N_NODES])
    m2 = _spmm(hcat, src2, dst, adj_edge_values)
    out = _dense(m2[0], m2[1], h0, h, W.T,
                 b[None, :], ln_weight[None, :], ln_bias[None, :])
    return out


# speedup vs baseline: 4.5828x; 1.7687x over previous
import functools

import jax
import jax.numpy as jnp
from jax import lax
from jax.experimental import pallas as pl
from jax.experimental.pallas import tpu as pltpu
from jax.experimental.pallas import tpu_sc as plsc

N_NODES = 10000
N_EDGES = 160000
DIM = 256
HD = 128
ALPHA = 0.1
BETA = 0.5

NS = 16
EPT = N_EDGES // NS
K = 80
CHUNKS = EPT // K
ROWS_PER_TILE = N_NODES // NS
ZR = 125


_GDN = lax.GatherDimensionNumbers(
    offset_dims=(), collapsed_slice_dims=(0,), start_index_map=(0,))


def _lane_bcast(vec16, j):
    return lax.gather(vec16, jnp.full((16, 1), j, jnp.int32), _GDN,
                      slice_sizes=(1,),
                      mode=lax.GatherScatterMode.PROMISE_IN_BOUNDS)


def _spmm_body(hcat_hbm, sv_hbm, dst_hbm, out_hbm, acc, zeros_v,
               sv_a, dst_a, rows_a, sv_b, dst_b, rows_b,
               gs_a, gs_b, ss_a, ss_b):
    c = lax.axis_index("c")
    s = lax.axis_index("s")

    z16 = jnp.zeros((16,), jnp.float32)

    def zrow(r, carry):
        for k in range(HD // 16):
            zeros_v[r, pl.ds(k * 16, 16)] = z16
        return carry

    lax.fori_loop(0, ZR, zrow, 0)
    for j in range(ROWS_PER_TILE // ZR):
        pltpu.sync_copy(zeros_v, acc.at[pl.ds(s * ROWS_PER_TILE + j * ZR, ZR)])
    plsc.subcore_barrier()

    sv_base = (c * NS + s) * CHUNKS
    dst_base = s * EPT

    def meta(i, sv, dst):
        pltpu.sync_copy(sv_hbm.at[pl.ds((sv_base + i) * 2 * K, 2 * K)], sv)
        pltpu.sync_copy(dst_hbm.at[pl.ds(dst_base + i * K, K)], dst)

    def g_start(sv, rows, sem):
        pltpu.async_copy(hcat_hbm.at[sv.at[pl.ds(0, K)]], rows, sem)

    def g_wait(sv, rows, sem):
        pltpu.make_async_copy(hcat_hbm.at[sv.at[pl.ds(0, K)]], rows,
                              sem).wait()

    def scat_start(rows, dst, sem):
        pltpu.async_copy(rows, acc.at[dst], sem, add=True)

    def scat_wait(rows, dst, sem):
        pltpu.make_async_copy(rows, acc.at[dst], sem).wait()

    def scale(sv, rows):
        def group(g, carry):
            val16 = lax.bitcast_convert_type(sv[pl.ds(K + g * 16, 16)],
                                             jnp.float32)
            for j in range(16):
                e = g * 16 + j
                vb = _lane_bcast(val16, j)
                for d in range(HD // 16):
                    x = rows[e, pl.ds(d * 16, 16)]
                    rows[e, pl.ds(d * 16, 16)] = x * vb
            return carry

        lax.fori_loop(0, K // 16, group, 0)

    meta(0, sv_a, dst_a)
    g_start(sv_a, rows_a, gs_a)
    meta(1, sv_b, dst_b)
    g_start(sv_b, rows_b, gs_b)

    def pipe(j, carry):
        c0 = 2 * j
        g_wait(sv_a, rows_a, gs_a)
        scale(sv_a, rows_a)
        scat_start(rows_a, dst_a, ss_a)

        g_wait(sv_b, rows_b, gs_b)
        scale(sv_b, rows_b)
        scat_start(rows_b, dst_b, ss_b)

        scat_wait(rows_a, dst_a, ss_a)
        meta(c0 + 2, sv_a, dst_a)
        g_start(sv_a, rows_a, gs_a)

        @pl.when(c0 + 3 < CHUNKS)
        def _refill_b():
            scat_wait(rows_b, dst_b, ss_b)
            meta(c0 + 3, sv_b, dst_b)
            g_start(sv_b, rows_b, gs_b)

        return carry

    lax.fori_loop(0, (CHUNKS - 1) // 2, pipe, 0)
    scat_wait(rows_b, dst_b, ss_b)
    g_wait(sv_a, rows_a, gs_a)
    scale(sv_a, rows_a)
    scat_start(rows_a, dst_a, ss_a)
    scat_wait(rows_a, dst_a, ss_a)

    plsc.subcore_barrier()
    pltpu.sync_copy(acc.at[pl.ds(s * 624, 624)],
                    out_hbm.at[c, pl.ds(s * 624, 624)])

    @pl.when(s == 0)
    def _tail():
        pltpu.sync_copy(acc.at[pl.ds(9984, 16)],
                        out_hbm.at[c, pl.ds(9984, 16)])


_spmm = functools.partial(
    pl.kernel,
    mesh=plsc.VectorSubcoreMesh(core_axis_name="c", subcore_axis_name="s"),
    out_type=jax.ShapeDtypeStruct((2, N_NODES, HD), jnp.float32),
    scratch_types=[
        pltpu.VMEM_SHARED((N_NODES, HD), jnp.float32),
        pltpu.VMEM((ZR, HD), jnp.float32),
        pltpu.VMEM((2 * K,), jnp.int32),
        pltpu.VMEM((K,), jnp.int32),
        pltpu.VMEM((K, HD), jnp.float32),
        pltpu.VMEM((2 * K,), jnp.int32),
        pltpu.VMEM((K,), jnp.int32),
        pltpu.VMEM((K, HD), jnp.float32),
        pltpu.SemaphoreType.DMA,
        pltpu.SemaphoreType.DMA,
        pltpu.SemaphoreType.DMA,
        pltpu.SemaphoreType.DMA,
    ],
)(_spmm_body)


_SQRT_HALF = 0.7071067811865476


def _erf(x):
    ax = jnp.abs(x)
    t = 1.0 / (1.0 + 0.3275911 * ax)
    poly = ((((1.061405429 * t - 1.453152027) * t + 1.421413741) * t
             - 0.284496736) * t + 0.254829592) * t
    e = 1.0 - poly * jnp.exp(-ax * ax)
    return jnp.sign(x) * e


def _dense_body(ma_ref, mb_ref, h0_ref, h_ref, wt_ref, b_ref, g_ref, be_ref,
                o_ref):
    mm = jnp.concatenate([ma_ref[...], mb_ref[...]], axis=1)
    mm = (1.0 - ALPHA) * mm + ALPHA * h0_ref[...]
    lin = jnp.dot(mm, wt_ref[...], preferred_element_type=jnp.float32)
    lin = lin + b_ref[...]
    x = (1.0 - BETA) * mm + BETA * lin
    g = 0.5 * x * (1.0 + _erf(x * _SQRT_HALF))
    y = g + h_ref[...]
    mean = jnp.mean(y, axis=1, keepdims=True)
    cen = y - mean
    var = jnp.mean(cen * cen, axis=1, keepdims=True)
    o_ref[...] = cen * lax.rsqrt(var + 1e-5) * g_ref[...] + be_ref[...]


R = 1000


def _dense(ma, mb, h0, h, wt, b2, g2, be2):
    grid = (N_NODES // R,)
    return pl.pallas_call(
        _dense_body,
        grid=grid,
        in_specs=[
            pl.BlockSpec((R, HD), lambda i: (i, 0)),
            pl.BlockSpec((R, HD), lambda i: (i, 0)),
            pl.BlockSpec((R, DIM), lambda i: (i, 0)),
            pl.BlockSpec((R, DIM), lambda i: (i, 0)),
            pl.BlockSpec((DIM, DIM), lambda i: (0, 0)),
            pl.BlockSpec((1, DIM), lambda i: (0, 0)),
            pl.BlockSpec((1, DIM), lambda i: (0, 0)),
            pl.BlockSpec((1, DIM), lambda i: (0, 0)),
        ],
        out_specs=pl.BlockSpec((R, DIM), lambda i: (i, 0)),
        out_shape=jax.ShapeDtypeStruct((N_NODES, DIM), jnp.float32),
    )(ma, mb, h0, h, wt, b2, g2, be2)


@jax.jit
def kernel(h, h0, adj_edge_index, adj_edge_values, W, b, ln_weight, ln_bias):
    src = adj_edge_index[0]
    dst = adj_edge_index[1]
    hcat = jnp.concatenate([h[:, :HD], h[:, HD:]], axis=0)
    src_chunks = src.reshape(NS * CHUNKS, K)
    valbits = lax.bitcast_convert_type(adj_edge_values,
                                       jnp.int32).reshape(NS * CHUNKS, K)
    sv = jnp.concatenate([
        jnp.concatenate([src_chunks, valbits], axis=1),
        jnp.concatenate([src_chunks + N_NODES, valbits], axis=1),
    ], axis=0).reshape(-1)
    m2 = _spmm(hcat, sv, dst)
    out = _dense(m2[0], m2[1], h0, h, W.T,
                 b[None, :], ln_weight[None, :], ln_bias[None, :])
    return out
